# Initial kernel scaffold; baseline (speedup 1.0000x reference)
import jax
import jax.numpy as jnp
from jax.experimental import pallas as pl


def kernel(state, W1, b1, W2, b2):
    raise NotImplementedError("write your pallas kernel here")


# speedup vs baseline: 2.9966x; 2.9966x over previous
import jax
import jax.numpy as jnp
from jax import lax
from jax.experimental import pallas as pl
from jax.experimental.pallas import tpu as pltpu
from jax.experimental.pallas import tpu_sc as plsc

STATE = 2048
NP = 32768
HID = 128
KTOP = 64
BATCH = 32

NPB = 4096
NBLK = NP // NPB

LANES = 16
NV = NP // LANES
CAP = 2048


def _mlp_body(state_ref, w1_ref, b1_ref, w2_ref, b2_ref, out_ref, h_ref):
    @pl.when(pl.program_id(0) == 0)
    def _():
        h = jnp.dot(state_ref[...], w1_ref[...],
                    preferred_element_type=jnp.float32)
        h_ref[...] = jnp.maximum(h + b1_ref[...], 0.0)

    out_ref[...] = jnp.dot(h_ref[...], w2_ref[...],
                           preferred_element_type=jnp.float32) + b2_ref[...]


def _logits(state, W1, b1, W2, b2):
    return pl.pallas_call(
        _mlp_body,
        grid=(NBLK,),
        in_specs=[
            pl.BlockSpec((BATCH, STATE), lambda i: (0, 0)),
            pl.BlockSpec((STATE, HID), lambda i: (0, 0)),
            pl.BlockSpec((1, HID), lambda i: (0, 0)),
            pl.BlockSpec((HID, NPB), lambda i: (0, i)),
            pl.BlockSpec((1, NPB), lambda i: (0, i)),
        ],
        out_specs=pl.BlockSpec((BATCH, NPB), lambda i: (0, i)),
        out_shape=jax.ShapeDtypeStruct((BATCH, NP), jnp.float32),
        scratch_shapes=[pltpu.VMEM((BATCH, HID), jnp.float32)],
    )(state, W1, b1.reshape(1, HID), W2, b2.reshape(1, NP))


def _topk_body(logits_hbm, out_hbm, row_v, cand_val, cand_idx, out_v):
    c = lax.axis_index("c")
    s = lax.axis_index("s")
    row = s * 2 + c

    pltpu.sync_copy(logits_hbm.at[row], row_v)

    neg_inf = jnp.float32(-jnp.inf)
    ninf_vec = jnp.full((LANES,), neg_inf, jnp.float32)

    def p1_body(i, carry):
        t1, t2, t3, t4 = carry
        v = row_v[pl.ds(i * LANES, LANES)]
        s1 = jnp.maximum(t1, v)
        r1 = jnp.minimum(t1, v)
        s2 = jnp.maximum(t2, r1)
        r2 = jnp.minimum(t2, r1)
        s3 = jnp.maximum(t3, r2)
        r3 = jnp.minimum(t3, r2)
        s4 = jnp.maximum(t4, r3)
        return s1, s2, s3, s4

    _, _, _, t4 = lax.fori_loop(
        0, NV, p1_body, (ninf_vec, ninf_vec, ninf_vec, ninf_vec))
    thr = jnp.min(t4)

    def fill_body(j, u):
        cand_val[pl.ds(j * LANES, LANES)] = ninf_vec
        return u

    lax.fori_loop(0, (CAP + LANES) // LANES, fill_body, 0)

    iota = lax.iota(jnp.int32, LANES)

    def p2_body(i, off):
        v = row_v[pl.ds(i * LANES, LANES)]
        m = v >= thr
        idx = iota + i * LANES
        plsc.store_compressed(cand_val.at[pl.ds(off, LANES)], v, mask=m)
        plsc.store_compressed(cand_idx.at[pl.ds(off, LANES)], idx, mask=m)
        return jnp.minimum(off + jnp.sum(m.astype(jnp.int32)),
                           jnp.int32(CAP))

    cnt = lax.fori_loop(0, NV, p2_body, jnp.int32(0))
    nvc = (cnt + LANES - 1) // LANES

    def sel_cond(carry):
        _, remaining, _, _ = carry
        return remaining > 0

    def sel_body(carry):
        thresh, remaining, v_star, n_eq = carry

        def mx_body(j, acc):
            v = cand_val[pl.ds(j * LANES, LANES)]
            return jnp.maximum(acc, jnp.where(v < thresh, v, neg_inf))

        m = jnp.max(lax.fori_loop(0, nvc, mx_body, ninf_vec))

        def ct_body(j, acc):
            v = cand_val[pl.ds(j * LANES, LANES)]
            return acc + jnp.sum((v == m).astype(jnp.int32))

        c_eq = lax.fori_loop(0, nvc, ct_body, jnp.int32(0))
        takes_all = c_eq < remaining
        new_rem = jnp.where(takes_all, remaining - c_eq, jnp.int32(0))
        v_star = jnp.where(takes_all, v_star, m)
        n_eq = jnp.where(takes_all, n_eq, remaining)
        return m, new_rem, v_star, n_eq

    _, _, v_star, n_eq = lax.while_loop(
        sel_cond, sel_body,
        (jnp.float32(jnp.inf), jnp.int32(KTOP),
         jnp.float32(jnp.inf), jnp.int32(0)))

    def fin_body(j, carry):
        off, eqs = carry
        v = cand_val[pl.ds(j * LANES, LANES)]
        idx = cand_idx[pl.ds(j * LANES, LANES)]
        gt = v > v_star
        eq = v == v_star
        pref = plsc.cumsum(eq.astype(jnp.int32)) + eqs
        take = gt | (eq & (pref <= n_eq))
        plsc.store_compressed(out_v.at[pl.ds(off, LANES)], idx, mask=take)
        return (off + jnp.sum(take.astype(jnp.int32)),
                eqs + jnp.sum(eq.astype(jnp.int32)))

    lax.fori_loop(0, nvc, fin_body, (jnp.int32(0), jnp.int32(0)))

    pltpu.sync_copy(out_v.at[pl.ds(0, KTOP)], out_hbm.at[pl.ds(row * KTOP, KTOP)])


def _topk(logits):
    mesh = plsc.VectorSubcoreMesh(core_axis_name="c", subcore_axis_name="s",
                                  num_cores=2, num_subcores=16)
    return pl.kernel(
        _topk_body,
        out_type=jax.ShapeDtypeStruct((BATCH * KTOP,), jnp.int32),
        mesh=mesh,
        scratch_types=[
            pltpu.VMEM((NP,), jnp.float32),
            pltpu.VMEM((CAP + LANES,), jnp.float32),
            pltpu.VMEM((CAP + LANES,), jnp.int32),
            pltpu.VMEM((KTOP + LANES,), jnp.int32),
        ],
        compiler_params=pltpu.CompilerParams(needs_layout_passes=False),
    )(logits)


def kernel(state, W1, b1, W2, b2):
    logits = _logits(state, W1, b1, W2, b2)
    indices = _topk(logits).reshape(BATCH, KTOP)
    return logits, indices
